# baseline (speedup 1.0000x reference)
import functools

import jax
import jax.numpy as jnp
from jax import lax
from jax.experimental import pallas as pl
from jax.experimental.pallas import tpu as pltpu
from jax.experimental.pallas import tpu_sc as plsc

VOCAB = 1000000
DIM = 32
B = 4096
L = 200

_INFO = plsc.get_sparse_core_info()
_NC = _INFO.num_cores
_NS = _INFO.num_subcores
_NW = _NC * _NS

_N = B * L
_CHUNK = 128
_CPB = B // _CHUNK
_PER_W = _N // _NW
_NCHUNK = _PER_W // _CHUNK

_mesh = plsc.VectorSubcoreMesh(core_axis_name="c", subcore_axis_name="s")


@functools.partial(
    pl.kernel,
    mesh=_mesh,
    out_type=jax.ShapeDtypeStruct((L, DIM, B), jnp.float32),
    scratch_types=[
        pltpu.VMEM((_NCHUNK, _CHUNK), jnp.int32),
        pltpu.VMEM((2, _CHUNK, DIM), jnp.float32),
        pltpu.VMEM((2, DIM, _CHUNK), jnp.float32),
        pltpu.SemaphoreType.DMA,
        pltpu.SemaphoreType.DMA,
    ],
    compiler_params=pltpu.CompilerParams(
        use_tc_tiling_on_sc=False, needs_layout_passes=False),
)
def _gather(weight_hbm, idx_hbm, out_hbm, idx_v, rows_v, t_v, gsem, ssem):
    wid = lax.axis_index("s") * _NC + lax.axis_index("c")
    base_chunk = wid * _NCHUNK

    pltpu.sync_copy(idx_hbm.at[pl.ds(base_chunk, _NCHUNK)], idx_v)

    def gather_start(j, buf):
        return pltpu.async_copy(weight_hbm.at[idx_v.at[j]], rows_v.at[buf], gsem)

    gather_start(0, 0)

    def body(j, _):
        buf = lax.rem(j, 2)
        pltpu.make_async_copy(weight_hbm.at[idx_v.at[j]], rows_v.at[buf], gsem).wait()
        @pl.when(j + 1 < _NCHUNK)
        def _():
            gather_start(j + 1, 1 - buf)

        src = rows_v.at[buf]
        dst = t_v.at[buf]

        def dbody(d, _):
            col = jax.lax.broadcast(d, (16,))
            for rb in range(_CHUNK // 16):
                rows_idx = jax.lax.iota(jnp.int32, 16) + rb * 16
                v = plsc.load_gather(src, [rows_idx, col])
                dst[d, pl.ds(rb * 16, 16)] = v
            return 0

        lax.fori_loop(0, DIM, dbody, 0)

        c = base_chunk + j
        l = c // _CPB
        bc = lax.rem(c, _CPB)
        store = pltpu.async_copy(
            dst, out_hbm.at[l, :, pl.ds(bc * _CHUNK, _CHUNK)], ssem)
        store.wait()
        return 0

    lax.fori_loop(0, _NCHUNK, body, 0)


def kernel(tensor, weight):
    idx_t = tensor.T.reshape(_N // _CHUNK, _CHUNK)
    out_t = _gather(weight, idx_t)
    return jnp.transpose(out_t, (0, 2, 1))


# speedup vs baseline: 1.2081x; 1.2081x over previous
import functools

import jax
import jax.numpy as jnp
from jax import lax
from jax.experimental import pallas as pl
from jax.experimental.pallas import tpu as pltpu
from jax.experimental.pallas import tpu_sc as plsc

VOCAB = 1000000
DIM = 32
B = 4096
L = 200

_INFO = plsc.get_sparse_core_info()
_NC = _INFO.num_cores
_NS = _INFO.num_subcores
_NW = _NC * _NS
_CHUNK = B // _NW

_mesh = plsc.VectorSubcoreMesh(core_axis_name="c", subcore_axis_name="s")


@functools.partial(
    pl.kernel,
    mesh=_mesh,
    out_type=jax.ShapeDtypeStruct((L, B, DIM), jnp.float32),
    scratch_types=[
        pltpu.VMEM((_CHUNK, L + 1), jnp.int32),
        pltpu.VMEM((2, _CHUNK), jnp.int32),
        pltpu.VMEM((2, _CHUNK, DIM), jnp.float32),
        pltpu.SemaphoreType.DMA,
        pltpu.SemaphoreType.DMA,
    ],
    compiler_params=pltpu.CompilerParams(
        use_tc_tiling_on_sc=False, needs_layout_passes=False),
)
def _gather(idx_hbm, weight_hbm, out_hbm, blk_v, idx_v, rows_v, gsem, ssem):
    wid = lax.axis_index("s") * _NC + lax.axis_index("c")
    b0 = wid * _CHUNK

    pltpu.sync_copy(idx_hbm.at[pl.ds(b0, _CHUNK)], blk_v.at[:, pl.ds(0, L)])

    def build_col(l):
        s = lax.rem(l, 2)
        col = jax.lax.broadcast(l, (16,))
        for k in range(_CHUNK // 16):
            rows = jax.lax.iota(jnp.int32, 16) + k * 16
            v = plsc.load_gather(blk_v, [rows, col])
            idx_v[s, pl.ds(k * 16, 16)] = v

    def gather_start(l, buf):
        pltpu.async_copy(
            weight_hbm.at[idx_v.at[lax.rem(l, 2)]], rows_v.at[buf], gsem)

    def gather_wait(l, buf):
        pltpu.make_async_copy(
            weight_hbm.at[idx_v.at[lax.rem(l, 2)]], rows_v.at[buf], gsem).wait()

    build_col(0)
    gather_start(0, 0)

    def body(l, _):
        buf = lax.rem(l, 2)

        @pl.when(l + 1 < L)
        def _():
            build_col(l + 1)

        gather_wait(l, buf)

        @pl.when(l + 1 < L)
        def _():
            gather_start(l + 1, 1 - buf)

        pltpu.async_copy(
            rows_v.at[buf], out_hbm.at[l, pl.ds(b0, _CHUNK)], ssem)
        pltpu.make_async_copy(
            rows_v.at[buf], out_hbm.at[l, pl.ds(b0, _CHUNK)], ssem).wait()
        return 0

    lax.fori_loop(0, L, body, 0)


def kernel(tensor, weight):
    return _gather(tensor, weight)


# speedup vs baseline: 1.4671x; 1.2144x over previous
import functools

import jax
import jax.numpy as jnp
from jax import lax
from jax.experimental import pallas as pl
from jax.experimental.pallas import tpu as pltpu
from jax.experimental.pallas import tpu_sc as plsc

VOCAB = 1000000
DIM = 32
B = 4096
L = 200

_INFO = plsc.get_sparse_core_info()
_NC = _INFO.num_cores
_NS = _INFO.num_subcores
_NW = _NC * _NS
_CHUNK = B // _NW
_TPITCH = _CHUNK + 8

_mesh = plsc.VectorSubcoreMesh(core_axis_name="c", subcore_axis_name="s")


@functools.partial(
    pl.kernel,
    mesh=_mesh,
    out_type=jax.ShapeDtypeStruct((L, DIM, B), jnp.float32),
    scratch_types=[
        pltpu.VMEM((_CHUNK, L + 1), jnp.int32),
        pltpu.VMEM((2, _CHUNK), jnp.int32),
        pltpu.VMEM((2, _CHUNK, DIM), jnp.float32),
        pltpu.VMEM((2, DIM, _TPITCH), jnp.float32),
        pltpu.SemaphoreType.DMA,
        pltpu.SemaphoreType.DMA,
    ],
    compiler_params=pltpu.CompilerParams(
        use_tc_tiling_on_sc=False, needs_layout_passes=False),
)
def _gather(idx_hbm, weight_hbm, out_hbm,
            blk_v, idx_v, rows_v, t_v, gsem, ssem):
    wid = lax.axis_index("s") * _NC + lax.axis_index("c")
    b0 = wid * _CHUNK

    pltpu.sync_copy(idx_hbm.at[pl.ds(b0, _CHUNK)], blk_v.at[:, pl.ds(0, L)])

    def build_col(l):
        s = lax.rem(l, 2)
        col = jax.lax.broadcast(l, (16,))
        for k in range(_CHUNK // 16):
            rows = jax.lax.iota(jnp.int32, 16) + k * 16
            v = plsc.load_gather(blk_v, [rows, col])
            idx_v[s, pl.ds(k * 16, 16)] = v

    def gather_start(l, buf):
        pltpu.async_copy(
            weight_hbm.at[idx_v.at[lax.rem(l, 2)]], rows_v.at[buf], gsem)

    def gather_wait(l, buf):
        pltpu.make_async_copy(
            weight_hbm.at[idx_v.at[lax.rem(l, 2)]], rows_v.at[buf], gsem).wait()

    def transpose(buf):
        src = rows_v.at[buf]
        dst = t_v.at[buf]
        d_lo = jax.lax.iota(jnp.int32, 16)
        d_hi = d_lo + 16

        def rbody(r, _):
            rr = jax.lax.broadcast(r, (16,))
            plsc.store_scatter(dst, [d_lo, rr], src[r, pl.ds(0, 16)])
            plsc.store_scatter(dst, [d_hi, rr], src[r, pl.ds(16, 16)])
            return 0

        lax.fori_loop(0, _CHUNK, rbody, 0)

    def store_start(l, buf):
        pltpu.async_copy(
            t_v.at[buf, :, pl.ds(0, _CHUNK)],
            out_hbm.at[l, :, pl.ds(b0, _CHUNK)], ssem)

    def store_wait(l, buf):
        pltpu.make_async_copy(
            t_v.at[buf, :, pl.ds(0, _CHUNK)],
            out_hbm.at[l, :, pl.ds(b0, _CHUNK)], ssem).wait()

    build_col(0)
    gather_start(0, 0)

    def body(l, _):
        buf = lax.rem(l, 2)

        @pl.when(l + 1 < L)
        def _():
            build_col(l + 1)

        gather_wait(l, buf)

        @pl.when(l + 1 < L)
        def _():
            gather_start(l + 1, 1 - buf)

        @pl.when(l >= 2)
        def _():
            store_wait(l - 2, buf)

        transpose(buf)
        store_start(l, buf)
        return 0

    lax.fori_loop(0, L, body, 0)
    store_wait(L - 2, 0)
    store_wait(L - 1, 1)


def kernel(tensor, weight):
    out_t = _gather(tensor, weight)
    return jnp.transpose(out_t, (0, 2, 1))


# speedup vs baseline: 1.5036x; 1.0248x over previous
import functools

import jax
import jax.numpy as jnp
from jax import lax
from jax.experimental import pallas as pl
from jax.experimental.pallas import tpu as pltpu
from jax.experimental.pallas import tpu_sc as plsc

VOCAB = 1000000
DIM = 32
B = 4096
L = 200

_INFO = plsc.get_sparse_core_info()
_NC = _INFO.num_cores
_NS = _INFO.num_subcores
_NW = _NC * _NS
_CHUNK = B // _NW
_TPITCH = _CHUNK + 8

_mesh = plsc.VectorSubcoreMesh(core_axis_name="c", subcore_axis_name="s")


@functools.partial(
    pl.kernel,
    mesh=_mesh,
    out_type=jax.ShapeDtypeStruct((L, DIM, B), jnp.float32),
    scratch_types=[
        pltpu.VMEM((_CHUNK, L + 1), jnp.int32),
        pltpu.VMEM((2, _CHUNK), jnp.int32),
        pltpu.VMEM((2, _CHUNK, DIM), jnp.float32),
        pltpu.VMEM((2, DIM, _TPITCH), jnp.float32),
        pltpu.SemaphoreType.DMA,
        pltpu.SemaphoreType.DMA,
    ],
    compiler_params=pltpu.CompilerParams(
        use_tc_tiling_on_sc=False, needs_layout_passes=False),
)
def _gather(idx_hbm, weight_hbm, out_hbm,
            blk_v, idx_v, rows_v, t_v, gsem, ssem):
    wid = lax.axis_index("s") * _NC + lax.axis_index("c")
    b0 = wid * _CHUNK

    pltpu.sync_copy(idx_hbm.at[pl.ds(b0, _CHUNK)], blk_v.at[:, pl.ds(0, L)])

    def build_col(l):
        s = lax.rem(l, 2)
        col = jax.lax.broadcast(l, (16,))
        for k in range(_CHUNK // 16):
            rows = jax.lax.iota(jnp.int32, 16) + k * 16
            v = plsc.load_gather(blk_v, [rows, col])
            idx_v[s, pl.ds(k * 16, 16)] = v

    def gather_start(l, buf):
        pltpu.async_copy(
            weight_hbm.at[idx_v.at[lax.rem(l, 2)]], rows_v.at[buf], gsem)

    def gather_wait(l, buf):
        pltpu.make_async_copy(
            weight_hbm.at[idx_v.at[lax.rem(l, 2)]], rows_v.at[buf], gsem).wait()

    def transpose(buf):
        src = rows_v.at[buf]
        dst = t_v.at[buf]
        d_lo = jax.lax.iota(jnp.int32, 16)
        d_hi = d_lo + 16

        def rbody(r4, _):
            r = r4 * 4
            for u in range(4):
                rr = jax.lax.broadcast(r + u, (16,))
                plsc.store_scatter(dst, [d_lo, rr], src[r + u, pl.ds(0, 16)])
                plsc.store_scatter(dst, [d_hi, rr], src[r + u, pl.ds(16, 16)])
            return 0

        lax.fori_loop(0, _CHUNK // 4, rbody, 0)

    def store_start(l, buf):
        pltpu.async_copy(
            t_v.at[buf, :, pl.ds(0, _CHUNK)],
            out_hbm.at[l, :, pl.ds(b0, _CHUNK)], ssem)

    def store_wait(l, buf):
        pltpu.make_async_copy(
            t_v.at[buf, :, pl.ds(0, _CHUNK)],
            out_hbm.at[l, :, pl.ds(b0, _CHUNK)], ssem).wait()

    build_col(0)
    gather_start(0, 0)

    def body(l, _):
        buf = lax.rem(l, 2)

        @pl.when(l + 1 < L)
        def _():
            build_col(l + 1)

        gather_wait(l, buf)

        @pl.when(l + 1 < L)
        def _():
            gather_start(l + 1, 1 - buf)

        @pl.when(l >= 2)
        def _():
            store_wait(l - 2, buf)

        transpose(buf)
        store_start(l, buf)
        return 0

    lax.fori_loop(0, L, body, 0)
    store_wait(L - 2, 0)
    store_wait(L - 1, 1)


def kernel(tensor, weight):
    out_t = _gather(tensor, weight)
    return jnp.transpose(out_t, (0, 2, 1))
